# baseline (speedup 1.0000x reference)
import functools

import jax
import jax.numpy as jnp
from jax import lax
from jax.experimental import pallas as pl
from jax.experimental.pallas import tpu as pltpu
from jax.experimental.pallas import tpu_sc as plsc

BATCH = 16384
EMBED_DIM = 32
NUM_TABLES = 4

_info = plsc.get_sparse_core_info()
_NC, _NS = _info.num_cores, _info.num_subcores
_NW = _NC * _NS
_BPW = BATCH // _NW


def _gather_concat_kernel(idx_hbm, u_hbm, r_hbm, o_hbm, p_hbm, out_hbm,
                          idx_v, rows_v, sem, wsem):
    wid = lax.axis_index("s") * _NC + lax.axis_index("c")
    base = wid * _BPW
    pltpu.sync_copy(idx_hbm.at[:, pl.ds(base, _BPW)], idx_v)
    tables = (u_hbm, r_hbm, o_hbm, p_hbm)
    copies = []
    for t in range(NUM_TABLES):
        copies.append(pltpu.async_copy(
            tables[t].at[idx_v.at[t]], rows_v.at[t], sem))
    for c in copies:
        c.wait()
    wcopies = []
    for t in range(NUM_TABLES):
        wcopies.append(pltpu.async_copy(
            rows_v.at[t],
            out_hbm.at[pl.ds(base, _BPW), pl.ds(t * EMBED_DIM, EMBED_DIM)],
            wsem))
    for c in wcopies:
        c.wait()


@jax.jit
def _run(idx_all, user_table, region_table, rating_table, product_table):
    mesh = plsc.VectorSubcoreMesh(core_axis_name="c", subcore_axis_name="s")
    return pl.kernel(
        _gather_concat_kernel,
        out_type=jax.ShapeDtypeStruct((BATCH, NUM_TABLES * EMBED_DIM),
                                      jnp.float32),
        mesh=mesh,
        scratch_types=[
            pltpu.VMEM((NUM_TABLES, _BPW), jnp.int32),
            pltpu.VMEM((NUM_TABLES, _BPW, EMBED_DIM), jnp.float32),
            pltpu.SemaphoreType.DMA,
            pltpu.SemaphoreType.DMA,
        ],
        compiler_params=pltpu.CompilerParams(use_tc_tiling_on_sc=False),
    )(idx_all, user_table, region_table, rating_table, product_table)


def kernel(reviewerID, region, overall, asin, user_table, region_table,
           rating_table, product_table):
    idx_all = jnp.stack([reviewerID.astype(jnp.int32),
                         region.astype(jnp.int32),
                         overall.astype(jnp.int32),
                         asin.astype(jnp.int32)])
    return _run(idx_all, user_table, region_table, rating_table,
                product_table)


# speedup vs baseline: 1.9588x; 1.9588x over previous
import jax
import jax.numpy as jnp
from jax import lax
from jax.experimental import pallas as pl
from jax.experimental.pallas import tpu as pltpu
from jax.experimental.pallas import tpu_sc as plsc

BATCH = 16384
EMBED_DIM = 32
NUM_TABLES = 4
NUM_SMALL = 77
SMALL_OFF = (0, 65, 71)

_info = plsc.get_sparse_core_info()
_NC, _NS = _info.num_cores, _info.num_subcores
_NW = _NC * _NS
_BPW = BATCH // _NW
_L = 16
_NG = _BPW // _L


def _gather_concat_kernel(idx_hbm, u_hbm, r_hbm, o_hbm, p_hbm, out_hbm,
                          idx_v, rows_v, small_v, sem, wsem):
    wid = lax.axis_index("s") * _NC + lax.axis_index("c")
    base = wid * _BPW
    pltpu.sync_copy(idx_hbm.at[:, pl.ds(base, _BPW)], idx_v)
    ucopy = pltpu.async_copy(u_hbm.at[idx_v.at[0]], rows_v.at[0], sem)
    pltpu.sync_copy(r_hbm, small_v.at[pl.ds(0, 65)])
    pltpu.sync_copy(o_hbm, small_v.at[pl.ds(65, 6)])
    pltpu.sync_copy(p_hbm, small_v.at[pl.ds(71, 6)])

    lanes = lax.iota(jnp.int32, _L)

    def group_body(g, carry):
        bvec = g * _L + lanes
        for t in range(1, NUM_TABLES):
            rvec = idx_v[t, pl.ds(g * _L, _L)] + SMALL_OFF[t - 1]
            tvec = jnp.full((_L,), t, jnp.int32)
            for c in range(EMBED_DIM):
                cvec = jnp.full((_L,), c, jnp.int32)
                vals = plsc.load_gather(small_v, [rvec, cvec])
                plsc.store_scatter(rows_v, [tvec, bvec, cvec], vals)
        return carry

    lax.fori_loop(0, _NG, group_body, 0)
    ucopy.wait()
    wcopies = []
    for t in range(NUM_TABLES):
        wcopies.append(pltpu.async_copy(
            rows_v.at[t],
            out_hbm.at[pl.ds(base, _BPW), pl.ds(t * EMBED_DIM, EMBED_DIM)],
            wsem))
    for c in wcopies:
        c.wait()


@jax.jit
def _run(idx_all, user_table, region_table, rating_table, product_table):
    mesh = plsc.VectorSubcoreMesh(core_axis_name="c", subcore_axis_name="s")
    return pl.kernel(
        _gather_concat_kernel,
        out_type=jax.ShapeDtypeStruct((BATCH, NUM_TABLES * EMBED_DIM),
                                      jnp.float32),
        mesh=mesh,
        scratch_types=[
            pltpu.VMEM((NUM_TABLES, _BPW), jnp.int32),
            pltpu.VMEM((NUM_TABLES, _BPW, EMBED_DIM), jnp.float32),
            pltpu.VMEM((NUM_SMALL, EMBED_DIM), jnp.float32),
            pltpu.SemaphoreType.DMA,
            pltpu.SemaphoreType.DMA,
        ],
        compiler_params=pltpu.CompilerParams(use_tc_tiling_on_sc=False,
                                             needs_layout_passes=False),
    )(idx_all, user_table, region_table, rating_table, product_table)


def kernel(reviewerID, region, overall, asin, user_table, region_table,
           rating_table, product_table):
    idx_all = jnp.stack([reviewerID.astype(jnp.int32),
                         region.astype(jnp.int32),
                         overall.astype(jnp.int32),
                         asin.astype(jnp.int32)])
    return _run(idx_all, user_table, region_table, rating_table,
                product_table)


# speedup vs baseline: 3.3796x; 1.7254x over previous
import jax
import jax.numpy as jnp
from jax import lax
from jax.experimental import pallas as pl
from jax.experimental.pallas import tpu as pltpu
from jax.experimental.pallas import tpu_sc as plsc

BATCH = 16384
EMBED_DIM = 32
NUM_TABLES = 4
NUM_SMALL = 77

_info = plsc.get_sparse_core_info()
_NC, _NS = _info.num_cores, _info.num_subcores
_NW = _NC * _NS
_BPW = BATCH // _NW


def _gather_concat_kernel(idx_hbm, u_hbm, r_hbm, o_hbm, p_hbm, out_hbm,
                          idx_v, rows_v, small_sh, sem, wsem):
    sid = lax.axis_index("s")
    wid = sid * _NC + lax.axis_index("c")
    base = wid * _BPW
    pltpu.sync_copy(idx_hbm.at[:, pl.ds(base, _BPW)], idx_v)
    ucopy = pltpu.async_copy(u_hbm.at[idx_v.at[0]], rows_v.at[0], sem)
    @pl.when(sid == 0)
    def _stage():
        pltpu.sync_copy(r_hbm, small_sh.at[pl.ds(0, 65)])
        pltpu.sync_copy(o_hbm, small_sh.at[pl.ds(65, 6)])
        pltpu.sync_copy(p_hbm, small_sh.at[pl.ds(71, 6)])
    plsc.subcore_barrier()
    copies = [ucopy]
    for t in range(1, NUM_TABLES):
        copies.append(pltpu.async_copy(
            small_sh.at[idx_v.at[t]], rows_v.at[t], sem))
    for c in copies:
        c.wait()
    wcopies = []
    for t in range(NUM_TABLES):
        wcopies.append(pltpu.async_copy(
            rows_v.at[t],
            out_hbm.at[pl.ds(base, _BPW), pl.ds(t * EMBED_DIM, EMBED_DIM)],
            wsem))
    for c in wcopies:
        c.wait()


@jax.jit
def _run(idx_all, user_table, region_table, rating_table, product_table):
    mesh = plsc.VectorSubcoreMesh(core_axis_name="c", subcore_axis_name="s")
    return pl.kernel(
        _gather_concat_kernel,
        out_type=jax.ShapeDtypeStruct((BATCH, NUM_TABLES * EMBED_DIM),
                                      jnp.float32),
        mesh=mesh,
        scratch_types=[
            pltpu.VMEM((NUM_TABLES, _BPW), jnp.int32),
            pltpu.VMEM((NUM_TABLES, _BPW, EMBED_DIM), jnp.float32),
            pltpu.VMEM_SHARED((NUM_SMALL, EMBED_DIM), jnp.float32),
            pltpu.SemaphoreType.DMA,
            pltpu.SemaphoreType.DMA,
        ],
        compiler_params=pltpu.CompilerParams(use_tc_tiling_on_sc=False,
                                             needs_layout_passes=False),
    )(idx_all, user_table, region_table, rating_table, product_table)


def kernel(reviewerID, region, overall, asin, user_table, region_table,
           rating_table, product_table):
    idx_all = jnp.stack([reviewerID.astype(jnp.int32),
                         region.astype(jnp.int32),
                         overall.astype(jnp.int32) + 65,
                         asin.astype(jnp.int32) + 71])
    return _run(idx_all, user_table, region_table, rating_table,
                product_table)


# speedup vs baseline: 3.3944x; 1.0044x over previous
import jax
import jax.numpy as jnp
from jax import lax
from jax.experimental import pallas as pl
from jax.experimental.pallas import tpu as pltpu
from jax.experimental.pallas import tpu_sc as plsc

BATCH = 16384
EMBED_DIM = 32
NUM_TABLES = 4

_info = plsc.get_sparse_core_info()
_NC, _NS = _info.num_cores, _info.num_subcores
_NW = _NC * _NS
_BPW = BATCH // _NW


def _gather_concat_kernel(rid_hbm, reg_hbm, ovr_hbm, asin_hbm,
                          u_hbm, r_hbm, o_hbm, p_hbm, out_hbm,
                          idx_v, rows_v, r_sh, o_sh, p_sh, sem, wsem):
    sid = lax.axis_index("s")
    wid = sid * _NC + lax.axis_index("c")
    base = wid * _BPW
    idx_hbms = (rid_hbm, reg_hbm, ovr_hbm, asin_hbm)
    icopies = [pltpu.async_copy(idx_hbms[t].at[pl.ds(base, _BPW)],
                                idx_v.at[t], sem)
               for t in range(NUM_TABLES)]
    icopies[0].wait()
    ucopy = pltpu.async_copy(u_hbm.at[idx_v.at[0]], rows_v.at[0], sem)
    @pl.when(sid == 0)
    def _stage():
        pltpu.sync_copy(r_hbm, r_sh)
        pltpu.sync_copy(o_hbm, o_sh)
        pltpu.sync_copy(p_hbm, p_sh)
    plsc.subcore_barrier()
    tables_sh = (r_sh, o_sh, p_sh)
    copies = [ucopy]
    for t in range(1, NUM_TABLES):
        icopies[t].wait()
        copies.append(pltpu.async_copy(
            tables_sh[t - 1].at[idx_v.at[t]], rows_v.at[t], sem))
    for c in copies:
        c.wait()
    wcopies = []
    for t in range(NUM_TABLES):
        wcopies.append(pltpu.async_copy(
            rows_v.at[t],
            out_hbm.at[pl.ds(base, _BPW), pl.ds(t * EMBED_DIM, EMBED_DIM)],
            wsem))
    for c in wcopies:
        c.wait()


@jax.jit
def _run(rid, reg, ovr, asin, user_table, region_table, rating_table,
         product_table):
    mesh = plsc.VectorSubcoreMesh(core_axis_name="c", subcore_axis_name="s")
    return pl.kernel(
        _gather_concat_kernel,
        out_type=jax.ShapeDtypeStruct((BATCH, NUM_TABLES * EMBED_DIM),
                                      jnp.float32),
        mesh=mesh,
        scratch_types=[
            pltpu.VMEM((NUM_TABLES, _BPW), jnp.int32),
            pltpu.VMEM((NUM_TABLES, _BPW, EMBED_DIM), jnp.float32),
            pltpu.VMEM_SHARED(region_table.shape, jnp.float32),
            pltpu.VMEM_SHARED(rating_table.shape, jnp.float32),
            pltpu.VMEM_SHARED(product_table.shape, jnp.float32),
            pltpu.SemaphoreType.DMA,
            pltpu.SemaphoreType.DMA,
        ],
        compiler_params=pltpu.CompilerParams(use_tc_tiling_on_sc=False,
                                             needs_layout_passes=False),
    )(rid, reg, ovr, asin, user_table, region_table, rating_table,
      product_table)


def kernel(reviewerID, region, overall, asin, user_table, region_table,
           rating_table, product_table):
    return _run(reviewerID.astype(jnp.int32), region.astype(jnp.int32),
                overall.astype(jnp.int32), asin.astype(jnp.int32),
                user_table, region_table, rating_table, product_table)
